# Initial kernel scaffold; baseline (speedup 1.0000x reference)
import jax
import jax.numpy as jnp
from jax.experimental import pallas as pl


def kernel(x, edge_index, batch, W_node, b_node, W_edge, b_edge, bn_gamma, bn_beta, W_conv, b_conv):
    raise NotImplementedError("write your pallas kernel here")


# speedup vs baseline: 16.3832x; 16.3832x over previous
import functools

import jax
import jax.numpy as jnp
from jax import lax
from jax.experimental import pallas as pl
from jax.experimental.pallas import tpu as pltpu
from jax.experimental.pallas import tpu_sc as plsc

_NC = 2
_NS = 16
_NW = _NC * _NS
_CH = 128


def _s1_body(x_ref, wnT_ref, bn_ref, weT_ref, be_ref, g_ref, bt_ref, wc_ref,
             natt_ref, da_ref, db_ref, h_ref):
    x = x_ref[...]
    nv = wnT_ref[0, :] - wnT_ref[1, :]
    H = x.shape[1]
    ev_s = weT_ref[0, :H] - weT_ref[1, :H]
    ev_d = weT_ref[0, H:] - weT_ref[1, H:]
    nlog = jnp.sum(x * nv[None, :], axis=1, keepdims=True)
    nlog = nlog + (bn_ref[0, 0] - bn_ref[0, 1])
    natt = 1.0 / (1.0 + jnp.exp(-nlog))
    da = jnp.sum(x * ev_s[None, :], axis=1, keepdims=True)
    db = jnp.sum(x * ev_d[None, :], axis=1, keepdims=True)
    db = db + (be_ref[0, 0] - be_ref[0, 1])
    xc = natt * x
    mean = jnp.mean(xc, axis=0, keepdims=True)
    var = jnp.mean((xc - mean) ** 2, axis=0, keepdims=True)
    xb = (xc - mean) * lax.rsqrt(var + 1e-5) * g_ref[...] + bt_ref[...]
    h_ref[...] = jnp.dot(xb, wc_ref[...], preferred_element_type=jnp.float32)
    natt_ref[...] = natt
    da_ref[...] = da
    db_ref[...] = db


def _s1(x, wnT, bn2, weT, be2, gamma, beta, wc):
    N, H = x.shape
    f32 = jnp.float32
    return pl.pallas_call(
        _s1_body,
        out_shape=[
            jax.ShapeDtypeStruct((N, 1), f32),
            jax.ShapeDtypeStruct((N, 1), f32),
            jax.ShapeDtypeStruct((N, 1), f32),
            jax.ShapeDtypeStruct((N, H), f32),
        ],
    )(x, wnT, bn2, weT, be2, gamma, beta, wc)


def _s2(src_pad, dst_pad, da, db, E, N):
    E_pad = src_pad.shape[0]
    EPW = E_pad // _NW
    f32, i32 = jnp.float32, jnp.int32
    mesh = plsc.VectorSubcoreMesh(core_axis_name="c", subcore_axis_name="s", num_cores=_NC, num_subcores=_NS)

    @functools.partial(
        pl.kernel,
        out_type=[
            jax.ShapeDtypeStruct((E_pad,), f32),
            jax.ShapeDtypeStruct((_NW * N,), f32),
        ],
        mesh=mesh,
        scratch_types=[
            pltpu.VMEM((EPW,), i32),
            pltpu.VMEM((EPW,), i32),
            pltpu.VMEM((N,), f32),
            pltpu.VMEM((N,), f32),
            pltpu.VMEM((N,), f32),
            pltpu.VMEM((EPW,), f32),
        ],
        compiler_params=pltpu.CompilerParams(needs_layout_passes=False),
    )
    def k(src_h, dst_h, da_h, db_h, ea_h, degp_h,
          src_v, dst_v, da_v, db_v, deg_v, ea_v):
        cid = lax.axis_index("c")
        sid = lax.axis_index("s")
        wid = cid * _NS + sid
        base = wid * EPW
        pltpu.sync_copy(src_h.at[pl.ds(base, EPW)], src_v)
        pltpu.sync_copy(dst_h.at[pl.ds(base, EPW)], dst_v)
        pltpu.sync_copy(da_h, da_v)
        pltpu.sync_copy(db_h, db_v)

        def zbody(i, c):
            deg_v[pl.ds(i * 16, 16)] = jnp.zeros((16,), f32)
            return c
        lax.fori_loop(0, N // 16, zbody, 0)

        iota = lax.broadcasted_iota(i32, (16,), 0)

        def body(j, c):
            off = j * 16
            sidx = src_v[pl.ds(off, 16)]
            didx = dst_v[pl.ds(off, 16)]
            a = plsc.load_gather(da_v, [sidx])
            b = plsc.load_gather(db_v, [didx])
            ea = 1.0 / (1.0 + jnp.exp(-(a + b)))
            g = base + off + iota
            ea = jnp.where(g < E, ea, 0.0)
            ea_v[pl.ds(off, 16)] = ea
            plsc.addupdate_scatter(deg_v, [sidx], ea)
            return c
        lax.fori_loop(0, EPW // 16, body, 0)

        pltpu.sync_copy(ea_v, ea_h.at[pl.ds(base, EPW)])
        pltpu.sync_copy(deg_v, degp_h.at[pl.ds(wid * N, N)])

    return k(src_pad, dst_pad, da, db)


def _s3_body(degp_ref, h_ref, dinv_ref, h2_ref):
    NW = degp_ref.shape[0]
    N, H = h_ref.shape
    ones = jnp.ones((NW, 1), jnp.float32)
    dc = lax.dot_general(degp_ref[...], ones, (((0,), (0,)), ((), ())),
                         preferred_element_type=jnp.float32)
    deg = dc + 1.0
    dinv = lax.rsqrt(deg)
    dinv_ref[...] = dinv
    h2_ref[pl.ds(0, N), :] = dinv * h_ref[...]
    h2_ref[pl.ds(N, h2_ref.shape[0] - N), :] = jnp.zeros(
        (h2_ref.shape[0] - N, H), jnp.float32)


def _s3(degp, h, N_pad):
    N, H = h.shape
    f32 = jnp.float32
    return pl.pallas_call(
        _s3_body,
        out_shape=[
            jax.ShapeDtypeStruct((N, 1), f32),
            jax.ShapeDtypeStruct((N_pad, H), f32),
        ],
    )(degp, h)


def _s4(h2p, src2, dst2, ea, N_pad, H):
    K = src2.shape[1]
    EPW = K * _CH
    f32, i32 = jnp.float32, jnp.int32
    RPT = N_pad // _NS
    NZ = RPT // _CH
    mesh = plsc.VectorSubcoreMesh(core_axis_name="c", subcore_axis_name="s", num_cores=_NC, num_subcores=_NS)

    @functools.partial(
        pl.kernel,
        out_type=jax.ShapeDtypeStruct((_NC, N_pad, H), f32),
        mesh=mesh,
        scratch_types=[
            pltpu.VMEM((K, _CH), i32),
            pltpu.VMEM((K, _CH), i32),
            pltpu.VMEM((EPW,), f32),
            pltpu.VMEM((_CH, H), f32),
            pltpu.VMEM_SHARED((N_pad, H), f32),
        ],
        compiler_params=pltpu.CompilerParams(needs_layout_passes=False),
    )
    def k(h2_h, src2_h, dst2_h, ea_h, accs_h,
          src_v, dst_v, ea_v, rows_v, acc_s):
        cid = lax.axis_index("c")
        sid = lax.axis_index("s")
        wid = cid * _NS + sid

        def zb(i, c):
            for t in range(H // 16):
                rows_v[i, pl.ds(t * 16, 16)] = jnp.zeros((16,), f32)
            return c
        lax.fori_loop(0, _CH, zb, 0)
        for t5 in range(NZ):
            pltpu.sync_copy(rows_v,
                            acc_s.at[pl.ds(sid * RPT + t5 * _CH, _CH)])
        plsc.subcore_barrier()

        pltpu.sync_copy(src2_h.at[wid], src_v)
        pltpu.sync_copy(dst2_h.at[wid], dst_v)
        pltpu.sync_copy(ea_h.at[pl.ds(wid * EPW, EPW)], ea_v)

        def chunk(kk, c):
            pltpu.sync_copy(h2_h.at[src_v.at[kk]], rows_v)

            def ebody(e, c2):
                eav = plsc.load_gather(
                    ea_v, [jnp.full((16,), kk * _CH + e, i32)])
                for t in range(H // 16):
                    sl = pl.ds(t * 16, 16)
                    rows_v[e, sl] = rows_v[e, sl] * eav
                return c2
            lax.fori_loop(0, _CH, ebody, 0)

            pltpu.sync_copy(rows_v, acc_s.at[dst_v.at[kk]], add=True)
            return c
        lax.fori_loop(0, K, chunk, 0)

        plsc.subcore_barrier()
        pltpu.sync_copy(acc_s.at[pl.ds(sid * RPT, RPT)],
                        accs_h.at[cid, pl.ds(sid * RPT, RPT)])

    return k(h2p, src2, dst2, ea)


def _s5_body(accs_ref, h2_ref, dinv_ref, bc_ref, batch_ref, pooled_ref):
    G = pooled_ref.shape[0]
    N = batch_ref.shape[0]
    acc = accs_ref[0, pl.ds(0, N), :] + accs_ref[1, pl.ds(0, N), :]
    h2 = h2_ref[pl.ds(0, N), :]
    agg = dinv_ref[...] * (acc + h2) + bc_ref[...]
    out = jnp.where(agg > 0, agg, jnp.exp(agg) - 1.0)
    gi = lax.broadcasted_iota(jnp.int32, (1, G), 1)
    oh = (batch_ref[...] == gi).astype(jnp.float32)
    pooled_ref[...] = lax.dot_general(
        oh, out, (((0,), (0,)), ((), ())),
        preferred_element_type=jnp.float32)


def _s5(accs, h2, dinv, bc, batch_col, G):
    H = h2.shape[1]
    return pl.pallas_call(
        _s5_body,
        out_shape=jax.ShapeDtypeStruct((G, H), jnp.float32),
    )(accs, h2, dinv, bc, batch_col)


def kernel(x, edge_index, batch, W_node, b_node, W_edge, b_edge,
           bn_gamma, bn_beta, W_conv, b_conv):
    N, H = x.shape
    E = edge_index.shape[1]
    G = 256
    i32 = jnp.int32

    quant = _NW * _CH
    E_pad = ((E + quant - 1) // quant) * quant
    EPW = E_pad // _NW
    K = EPW // _CH
    pad = E_pad - E
    src = jnp.concatenate([edge_index[0], jnp.zeros((pad,), i32)])
    dst = jnp.concatenate([edge_index[1], jnp.zeros((pad,), i32)])

    natt, da, db, h = _s1(
        x, W_node.T, b_node.reshape(1, 2), W_edge.T, b_edge.reshape(1, 2),
        bn_gamma.reshape(1, H), bn_beta.reshape(1, H), W_conv)

    ea, degp = _s2(src, dst, da.reshape(N), db.reshape(N), E, N)

    N_pad = _NS * _CH * ((N + _NS * _CH - 1) // (_NS * _CH))
    dinv, h2p = _s3(degp.reshape(_NW, N), h, N_pad)

    accs = _s4(h2p, src.reshape(_NW, K, _CH), dst.reshape(_NW, K, _CH),
               ea, N_pad, H)

    pooled = _s5(accs, h2p, dinv, b_conv.reshape(1, H),
                 batch.reshape(N, 1), G)

    return (pooled, ea[:E], natt)
